# baseline (speedup 1.0000x reference)
import functools

import jax
import jax.numpy as jnp
from jax import lax
from jax.experimental import pallas as pl
from jax.experimental.pallas import tpu as pltpu
from jax.experimental.pallas import tpu_sc as plsc

_NC = 2
_NS = 16
_NW = _NC * _NS
_L = 16


def _sc_argmax_gather(card_prob, card_embed):
    B, V = card_prob.shape
    _, D = card_embed.shape
    bpw = B // _NW
    n_chunks = 4
    ch = V // n_chunks

    mesh = plsc.VectorSubcoreMesh(core_axis_name="c", subcore_axis_name="s")

    @functools.partial(
        pl.kernel,
        out_type=(
            jax.ShapeDtypeStruct((_NW, _L), jnp.int32),
            jax.ShapeDtypeStruct((_NW, _L), jnp.float32),
            jax.ShapeDtypeStruct((B, D), jnp.float32),
        ),
        mesh=mesh,
        compiler_params=pltpu.CompilerParams(use_tc_tiling_on_sc=False),
        scratch_types=(
            pltpu.VMEM((bpw, ch), jnp.float32),
            pltpu.VMEM((_L,), jnp.int32),
            pltpu.VMEM((_L,), jnp.float32),
            pltpu.VMEM((_L, D), jnp.float32),
            pltpu.SemaphoreType.DMA,
        ),
    )
    def sc_kernel(prob_hbm, embed_hbm, idx_out, val_out, card_out,
                  buf, idxv, valv, rowsv, sem):
        cid = lax.axis_index("c")
        sid = lax.axis_index("s")
        wid = sid * _NC + cid
        base = wid * bpw

        lanes = lax.iota(jnp.int32, _L)
        neg = jnp.full((_L,), -jnp.inf, jnp.float32)
        zero_i = jnp.zeros((_L,), jnp.int32)

        mv = [neg for _ in range(bpw)]
        mi = [zero_i for _ in range(bpw)]

        for c in range(n_chunks):
            pltpu.sync_copy(
                prob_hbm.at[pl.ds(base, bpw), pl.ds(c * ch, ch)], buf)

            def chunk_body(i, carry):
                col = carry[0]
                vals = list(carry[1])
                idxs = list(carry[2])
                for r in range(bpw):
                    v = buf[r, pl.ds(i * _L, _L)]
                    gt = v > vals[r]
                    vals[r] = jnp.where(gt, v, vals[r])
                    idxs[r] = jnp.where(gt, col, idxs[r])
                return (col + _L, tuple(vals), tuple(idxs))

            col0 = lanes + c * ch
            _, mvt, mit = lax.fori_loop(
                0, ch // _L, chunk_body, (col0, tuple(mv), tuple(mi)))
            mv = list(mvt)
            mi = list(mit)

        acc_i = zero_i
        acc_v = jnp.zeros((_L,), jnp.float32)
        for r in range(bpw):
            best_v = mv[r][0]
            best_i = mi[r][0]
            for lane in range(1, _L):
                v = mv[r][lane]
                i = mi[r][lane]
                take = (v > best_v) | ((v == best_v) & (i < best_i))
                best_v = jnp.where(take, v, best_v)
                best_i = jnp.where(take, i, best_i)
            acc_i = jnp.where(lanes == r, best_i, acc_i)
            acc_v = jnp.where(lanes == r, best_v, acc_v)

        idxv[...] = acc_i
        valv[...] = acc_v
        pltpu.async_copy(embed_hbm.at[idxv], rowsv, sem).wait()

        pltpu.sync_copy(idxv, idx_out.at[wid])
        pltpu.sync_copy(valv, val_out.at[wid])
        pltpu.sync_copy(rowsv.at[pl.ds(0, bpw)], card_out.at[pl.ds(base, bpw)])

    return sc_kernel(card_prob, card_embed)


def _tc_pos_stage(choice_card, pos_x_vector, pos_y_vector):
    B, P, D = pos_x_vector.shape
    pb = 128
    npb = P // pb
    def body(card_ref, x_ref, y_ref, px_ref, pxp_ref, py_ref, pyp_ref,
             mx_s, sx_s, ax_s, my_s, sy_s, ay_s):
        pi = pl.program_id(0)
        card = card_ref[...]

        def process(ref, m_s, s_s, a_s):
            blk = ref[...]
            logits = jnp.sum(blk * card[:, None, :], axis=-1)
            bm = jnp.max(logits, axis=-1, keepdims=True)
            col = lax.broadcasted_iota(jnp.int32, (B, pb), 1) + pi * pb
            barg = jnp.min(jnp.where(logits == bm, col, 2**30),
                           axis=-1, keepdims=True)
            bs = jnp.sum(jnp.exp(logits - bm), axis=-1, keepdims=True)

            @pl.when(pi == 0)
            def _():
                m_s[...] = bm
                s_s[...] = bs
                a_s[...] = barg

            @pl.when(pi > 0)
            def _():
                m_old = m_s[...]
                m_new = jnp.maximum(m_old, bm)
                s_s[...] = (s_s[...] * jnp.exp(m_old - m_new)
                            + bs * jnp.exp(bm - m_new))
                a_s[...] = jnp.where(bm > m_old, barg, a_s[...])
                m_s[...] = m_new

        process(x_ref, mx_s, sx_s, ax_s)
        process(y_ref, my_s, sy_s, ay_s)

        @pl.when(pi == npb - 1)
        def _():
            px_ref[...] = ax_s[...]
            pxp_ref[...] = 1.0 / sx_s[...]
            py_ref[...] = ay_s[...]
            pyp_ref[...] = 1.0 / sy_s[...]

    return pl.pallas_call(
        body,
        grid=(npb,),
        in_specs=[
            pl.BlockSpec((B, D), lambda i: (0, 0)),
            pl.BlockSpec((B, pb, D), lambda i: (0, i, 0)),
            pl.BlockSpec((B, pb, D), lambda i: (0, i, 0)),
        ],
        out_specs=[
            pl.BlockSpec((B, 1), lambda i: (0, 0)),
            pl.BlockSpec((B, 1), lambda i: (0, 0)),
            pl.BlockSpec((B, 1), lambda i: (0, 0)),
            pl.BlockSpec((B, 1), lambda i: (0, 0)),
        ],
        out_shape=[
            jax.ShapeDtypeStruct((B, 1), jnp.int32),
            jax.ShapeDtypeStruct((B, 1), jnp.float32),
            jax.ShapeDtypeStruct((B, 1), jnp.int32),
            jax.ShapeDtypeStruct((B, 1), jnp.float32),
        ],
        scratch_shapes=[
            pltpu.VMEM((B, 1), jnp.float32),
            pltpu.VMEM((B, 1), jnp.float32),
            pltpu.VMEM((B, 1), jnp.int32),
            pltpu.VMEM((B, 1), jnp.float32),
            pltpu.VMEM((B, 1), jnp.float32),
            pltpu.VMEM((B, 1), jnp.int32),
        ],
    )(choice_card, pos_x_vector, pos_y_vector)


def kernel(card_prob, pos_x_vector, pos_y_vector, card_embed, skip):
    B, V = card_prob.shape
    bpw = B // _NW

    idx_nw, val_nw, rows = _sc_argmax_gather(card_prob, card_embed)
    raw_idx = idx_nw[:, :bpw].reshape(B)
    raw_val = val_nw[:, :bpw].reshape(B)

    skip_flag = jnp.asarray(skip, jnp.int32) != 0
    choice_index = jnp.where(skip_flag, jnp.int32(0), raw_idx).astype(jnp.int32)
    card_prob_sel = jnp.where(skip_flag, card_prob[:, 0], raw_val)
    choice_card = jnp.where(skip_flag, card_embed[0][None, :], rows)

    px, pxp, py, pyp = _tc_pos_stage(choice_card, pos_x_vector, pos_y_vector)

    return (
        choice_index,
        card_prob_sel,
        px[:, 0],
        pxp[:, 0],
        py[:, 0],
        pyp[:, 0],
        choice_card,
    )


# speedup vs baseline: 3.7850x; 3.7850x over previous
import functools

import jax
import jax.numpy as jnp
from jax import lax
from jax.experimental import pallas as pl
from jax.experimental.pallas import tpu as pltpu
from jax.experimental.pallas import tpu_sc as plsc

_NC = 2
_NS = 16
_NW = _NC * _NS
_L = 16


def _sc_argmax_gather(card_prob, card_embed):
    B, V = card_prob.shape
    _, D = card_embed.shape
    bpw = B // _NW
    n_chunks = 4
    ch = V // n_chunks

    mesh = plsc.VectorSubcoreMesh(core_axis_name="c", subcore_axis_name="s")

    @functools.partial(
        pl.kernel,
        out_type=(
            jax.ShapeDtypeStruct((_NW, _L), jnp.int32),
            jax.ShapeDtypeStruct((_NW, _L), jnp.float32),
            jax.ShapeDtypeStruct((B, D), jnp.float32),
        ),
        mesh=mesh,
        compiler_params=pltpu.CompilerParams(use_tc_tiling_on_sc=False),
        scratch_types=(
            pltpu.VMEM((bpw, ch), jnp.float32),
            pltpu.VMEM((_L,), jnp.int32),
            pltpu.VMEM((_L,), jnp.float32),
            pltpu.VMEM((_L, D), jnp.float32),
            pltpu.SemaphoreType.DMA,
        ),
    )
    def sc_kernel(prob_hbm, embed_hbm, idx_out, val_out, card_out,
                  buf, idxv, valv, rowsv, sem):
        cid = lax.axis_index("c")
        sid = lax.axis_index("s")
        wid = sid * _NC + cid
        base = wid * bpw

        lanes = lax.iota(jnp.int32, _L)
        neg = jnp.full((_L,), -jnp.inf, jnp.float32)
        zero_i = jnp.zeros((_L,), jnp.int32)

        mv = [neg for _ in range(bpw)]
        mi = [zero_i for _ in range(bpw)]

        for c in range(n_chunks):
            pltpu.sync_copy(
                prob_hbm.at[pl.ds(base, bpw), pl.ds(c * ch, ch)], buf)

            def chunk_body(i, carry):
                col = carry[0]
                vals = list(carry[1])
                idxs = list(carry[2])
                for r in range(bpw):
                    v = buf[r, pl.ds(i * _L, _L)]
                    gt = v > vals[r]
                    vals[r] = jnp.where(gt, v, vals[r])
                    idxs[r] = jnp.where(gt, col, idxs[r])
                return (col + _L, tuple(vals), tuple(idxs))

            col0 = lanes + c * ch
            _, mvt, mit = lax.fori_loop(
                0, ch // _L, chunk_body, (col0, tuple(mv), tuple(mi)))
            mv = list(mvt)
            mi = list(mit)

        acc_i = zero_i
        acc_v = jnp.zeros((_L,), jnp.float32)
        for r in range(bpw):
            best_v = mv[r][0]
            best_i = mi[r][0]
            for lane in range(1, _L):
                v = mv[r][lane]
                i = mi[r][lane]
                take = (v > best_v) | ((v == best_v) & (i < best_i))
                best_v = jnp.where(take, v, best_v)
                best_i = jnp.where(take, i, best_i)
            acc_i = jnp.where(lanes == r, best_i, acc_i)
            acc_v = jnp.where(lanes == r, best_v, acc_v)

        idxv[...] = acc_i
        valv[...] = acc_v
        pltpu.async_copy(embed_hbm.at[idxv], rowsv, sem).wait()

        pltpu.sync_copy(idxv, idx_out.at[wid])
        pltpu.sync_copy(valv, val_out.at[wid])
        pltpu.sync_copy(rowsv.at[pl.ds(0, bpw)], card_out.at[pl.ds(base, bpw)])

    return sc_kernel(card_prob, card_embed)


def _tc_pos_stage(choice_card, pos_xt, pos_yt):
    B, D, P = pos_xt.shape
    bb = 16
    nbb = B // bb

    def body(card_ref, x_ref, y_ref, px_ref, pxp_ref, py_ref, pyp_ref):
        card = card_ref[...]

        def process(ref, out_i_ref, out_p_ref):
            blk = ref[...]
            lg = jnp.sum(blk * card[:, :, None], axis=1)
            bm = jnp.max(lg, axis=-1, keepdims=True)
            col = lax.broadcasted_iota(jnp.int32, (bb, P), 1)
            barg = jnp.min(jnp.where(lg == bm, col, 2**30),
                           axis=-1, keepdims=True)
            ssum = jnp.sum(jnp.exp(lg - bm), axis=-1, keepdims=True)
            out_i_ref[...] = barg
            out_p_ref[...] = 1.0 / ssum

        process(x_ref, px_ref, pxp_ref)
        process(y_ref, py_ref, pyp_ref)

    return pl.pallas_call(
        body,
        grid=(nbb,),
        in_specs=[
            pl.BlockSpec((bb, D), lambda i: (i, 0)),
            pl.BlockSpec((bb, D, P), lambda i: (i, 0, 0)),
            pl.BlockSpec((bb, D, P), lambda i: (i, 0, 0)),
        ],
        out_specs=[
            pl.BlockSpec((bb, 1), lambda i: (i, 0)),
            pl.BlockSpec((bb, 1), lambda i: (i, 0)),
            pl.BlockSpec((bb, 1), lambda i: (i, 0)),
            pl.BlockSpec((bb, 1), lambda i: (i, 0)),
        ],
        out_shape=[
            jax.ShapeDtypeStruct((B, 1), jnp.int32),
            jax.ShapeDtypeStruct((B, 1), jnp.float32),
            jax.ShapeDtypeStruct((B, 1), jnp.int32),
            jax.ShapeDtypeStruct((B, 1), jnp.float32),
        ],
    )(choice_card, pos_xt, pos_yt)


def kernel(card_prob, pos_x_vector, pos_y_vector, card_embed, skip):
    B, V = card_prob.shape
    bpw = B // _NW

    idx_nw, val_nw, rows = _sc_argmax_gather(card_prob, card_embed)
    raw_idx = idx_nw[:, :bpw].reshape(B)
    raw_val = val_nw[:, :bpw].reshape(B)

    skip_flag = jnp.asarray(skip, jnp.int32) != 0
    choice_index = jnp.where(skip_flag, jnp.int32(0), raw_idx).astype(jnp.int32)
    card_prob_sel = jnp.where(skip_flag, card_prob[:, 0], raw_val)
    choice_card = jnp.where(skip_flag, card_embed[0][None, :], rows)

    pos_xt = jnp.transpose(pos_x_vector, (0, 2, 1))
    pos_yt = jnp.transpose(pos_y_vector, (0, 2, 1))
    px, pxp, py, pyp = _tc_pos_stage(choice_card, pos_xt, pos_yt)

    return (
        choice_index,
        card_prob_sel,
        px[:, 0],
        pxp[:, 0],
        py[:, 0],
        pyp[:, 0],
        choice_card,
    )


# speedup vs baseline: 4.0817x; 1.0784x over previous
import functools

import jax
import jax.numpy as jnp
from jax import lax
from jax.experimental import pallas as pl
from jax.experimental.pallas import tpu as pltpu
from jax.experimental.pallas import tpu_sc as plsc

_NC = 2
_NS = 16
_NW = _NC * _NS
_L = 16


def _sc_argmax_gather(prob_flat, embed_flat, B, V, D):
    bpw = B // _NW
    n_chunks = 4
    ch = V // n_chunks

    mesh = plsc.VectorSubcoreMesh(core_axis_name="c", subcore_axis_name="s")

    @functools.partial(
        pl.kernel,
        out_type=(
            jax.ShapeDtypeStruct((_NW, _L), jnp.int32),
            jax.ShapeDtypeStruct((_NW, _L), jnp.float32),
            jax.ShapeDtypeStruct((B, D), jnp.float32),
        ),
        mesh=mesh,
        compiler_params=pltpu.CompilerParams(use_tc_tiling_on_sc=False),
        scratch_types=(
            pltpu.VMEM((bpw, ch), jnp.float32),
            pltpu.VMEM((bpw, ch), jnp.float32),
            pltpu.VMEM((_L,), jnp.int32),
            pltpu.VMEM((_L,), jnp.float32),
            pltpu.VMEM((bpw, D), jnp.float32),
            pltpu.SemaphoreType.DMA,
            pltpu.SemaphoreType.DMA,
        ),
    )
    def sc_kernel(prob_hbm, embed_hbm, idx_out, val_out, card_out,
                  buf_a, buf_b, idxv, valv, rowsv, sem_a, sem_b):
        cid = lax.axis_index("c")
        sid = lax.axis_index("s")
        wid = sid * _NC + cid
        base = wid * bpw

        lanes = lax.iota(jnp.int32, _L)
        neg = jnp.full((_L,), -jnp.inf, jnp.float32)
        zero_i = jnp.zeros((_L,), jnp.int32)

        bufs = (buf_a, buf_b)
        sems = (sem_a, sem_b)

        def start_chunk(c):
            buf, sem = bufs[c % 2], sems[c % 2]
            return [
                pltpu.async_copy(
                    prob_hbm.at[pl.ds((base + r) * V + c * ch, ch)],
                    buf.at[r], sem)
                for r in range(bpw)
            ]

        mv = [neg for _ in range(bpw)]
        mi = [zero_i for _ in range(bpw)]

        pending = start_chunk(0)
        UNROLL = 4
        for c in range(n_chunks):
            nxt = start_chunk(c + 1) if c + 1 < n_chunks else []
            for cp in pending:
                cp.wait()
            pending = nxt
            buf = bufs[c % 2]

            def chunk_body(i, carry, buf=buf):
                col = carry[0]
                vals = list(carry[1])
                idxs = list(carry[2])
                for u in range(UNROLL):
                    cu = col + u * _L
                    for r in range(bpw):
                        v = buf[r, pl.ds((i * UNROLL + u) * _L, _L)]
                        gt = v > vals[r]
                        vals[r] = jnp.where(gt, v, vals[r])
                        idxs[r] = jnp.where(gt, cu, idxs[r])
                return (col + UNROLL * _L, tuple(vals), tuple(idxs))

            col0 = lanes + c * ch
            _, mvt, mit = lax.fori_loop(
                0, ch // (_L * UNROLL), chunk_body, (col0, tuple(mv), tuple(mi)))
            mv = list(mvt)
            mi = list(mit)

        acc_i = zero_i
        acc_v = jnp.zeros((_L,), jnp.float32)
        for r in range(bpw):
            best_v = mv[r][0]
            best_i = mi[r][0]
            for lane in range(1, _L):
                v = mv[r][lane]
                i = mi[r][lane]
                take = (v > best_v) | ((v == best_v) & (i < best_i))
                best_v = jnp.where(take, v, best_v)
                best_i = jnp.where(take, i, best_i)
            pltpu.sync_copy(embed_flat_ref_at(embed_hbm, best_i), rowsv.at[r])
            acc_i = jnp.where(lanes == r, best_i, acc_i)
            acc_v = jnp.where(lanes == r, best_v, acc_v)

        idxv[...] = acc_i
        valv[...] = acc_v

        pltpu.sync_copy(idxv, idx_out.at[wid])
        pltpu.sync_copy(valv, val_out.at[wid])
        pltpu.sync_copy(rowsv, card_out.at[pl.ds(base, bpw)])

    def embed_flat_ref_at(embed_hbm, row_idx):
        return embed_hbm.at[pl.ds(row_idx * D, D)]

    return sc_kernel(prob_flat, embed_flat)


def _tc_pos_stage(choice_card, pos_xt, pos_yt):
    B, D, P = pos_xt.shape
    bb = 16
    nbb = B // bb

    def body(card_ref, x_ref, y_ref, px_ref, pxp_ref, py_ref, pyp_ref):
        card = card_ref[...]

        def process(ref, out_i_ref, out_p_ref):
            blk = ref[...]
            lg = jnp.sum(blk * card[:, :, None], axis=1)
            bm = jnp.max(lg, axis=-1, keepdims=True)
            col = lax.broadcasted_iota(jnp.int32, (bb, P), 1)
            barg = jnp.min(jnp.where(lg == bm, col, 2**30),
                           axis=-1, keepdims=True)
            ssum = jnp.sum(jnp.exp(lg - bm), axis=-1, keepdims=True)
            out_i_ref[...] = barg
            out_p_ref[...] = 1.0 / ssum

        process(x_ref, px_ref, pxp_ref)
        process(y_ref, py_ref, pyp_ref)

    return pl.pallas_call(
        body,
        grid=(nbb,),
        in_specs=[
            pl.BlockSpec((bb, D), lambda i: (i, 0)),
            pl.BlockSpec((bb, D, P), lambda i: (i, 0, 0)),
            pl.BlockSpec((bb, D, P), lambda i: (i, 0, 0)),
        ],
        out_specs=[
            pl.BlockSpec((bb, 1), lambda i: (i, 0)),
            pl.BlockSpec((bb, 1), lambda i: (i, 0)),
            pl.BlockSpec((bb, 1), lambda i: (i, 0)),
            pl.BlockSpec((bb, 1), lambda i: (i, 0)),
        ],
        out_shape=[
            jax.ShapeDtypeStruct((B, 1), jnp.int32),
            jax.ShapeDtypeStruct((B, 1), jnp.float32),
            jax.ShapeDtypeStruct((B, 1), jnp.int32),
            jax.ShapeDtypeStruct((B, 1), jnp.float32),
        ],
    )(choice_card, pos_xt, pos_yt)


def kernel(card_prob, pos_x_vector, pos_y_vector, card_embed, skip):
    B, V = card_prob.shape
    bpw = B // _NW

    _, D = card_embed.shape
    idx_nw, val_nw, rows = _sc_argmax_gather(
        card_prob.reshape(-1), card_embed.reshape(-1), B, V, D)
    raw_idx = idx_nw[:, :bpw].reshape(B)
    raw_val = val_nw[:, :bpw].reshape(B)

    skip_flag = jnp.asarray(skip, jnp.int32) != 0
    choice_index = jnp.where(skip_flag, jnp.int32(0), raw_idx).astype(jnp.int32)
    card_prob_sel = jnp.where(skip_flag, card_prob[:, 0], raw_val)
    choice_card = jnp.where(skip_flag, card_embed[0][None, :], rows)

    pos_xt = jnp.transpose(pos_x_vector, (0, 2, 1))
    pos_yt = jnp.transpose(pos_y_vector, (0, 2, 1))
    px, pxp, py, pyp = _tc_pos_stage(choice_card, pos_xt, pos_yt)

    return (
        choice_index,
        card_prob_sel,
        px[:, 0],
        pxp[:, 0],
        py[:, 0],
        pyp[:, 0],
        choice_card,
    )


# speedup vs baseline: 4.9501x; 1.2128x over previous
import functools

import jax
import jax.numpy as jnp
from jax import lax
from jax.experimental import pallas as pl
from jax.experimental.pallas import tpu as pltpu
from jax.experimental.pallas import tpu_sc as plsc

_NC = 2
_NS = 16
_NW = _NC * _NS
_L = 16


def _sc_card_argmax(card_prob, B, V):
    rpw = 8
    ngrp = B // rpw
    nh = _NW // ngrp
    vh = V // nh
    cw = 2048
    n_chunks = vh // cw

    mesh = plsc.VectorSubcoreMesh(core_axis_name="c", subcore_axis_name="s")

    @functools.partial(
        pl.kernel,
        out_type=(
            jax.ShapeDtypeStruct((_NW, _L), jnp.int32),
            jax.ShapeDtypeStruct((_NW, _L), jnp.float32),
        ),
        mesh=mesh,
        compiler_params=pltpu.CompilerParams(use_tc_tiling_on_sc=True),
        scratch_types=(
            pltpu.VMEM((rpw, cw), jnp.float32),
            pltpu.VMEM((rpw, cw), jnp.float32),
            pltpu.VMEM((_L,), jnp.int32),
            pltpu.VMEM((_L,), jnp.float32),
            pltpu.SemaphoreType.DMA,
            pltpu.SemaphoreType.DMA,
        ),
    )
    def sc_kernel(prob_hbm, idx_out, val_out, buf_a, buf_b, idxv, valv,
                  sem_a, sem_b):
        cid = lax.axis_index("c")
        sid = lax.axis_index("s")
        wid = sid * _NC + cid
        grp = wid // nh
        half = wid % nh
        row0 = grp * rpw
        col_base = half * vh

        lanes = lax.iota(jnp.int32, _L)
        neg = jnp.full((_L,), -jnp.inf, jnp.float32)
        zero_i = jnp.zeros((_L,), jnp.int32)

        bufs = (buf_a, buf_b)
        sems = (sem_a, sem_b)

        def start_chunk(c):
            return pltpu.async_copy(
                prob_hbm.at[pl.ds(row0, rpw),
                            pl.ds(col_base + c * cw, cw)],
                bufs[c % 2], sems[c % 2])

        mv = [neg for _ in range(rpw)]
        mi = [zero_i for _ in range(rpw)]

        pending = start_chunk(0)
        UNROLL = 2
        for c in range(n_chunks):
            nxt = start_chunk(c + 1) if c + 1 < n_chunks else None
            pending.wait()
            pending = nxt
            buf = bufs[c % 2]

            def chunk_body(i, carry, buf=buf):
                col = carry[0]
                vals = list(carry[1])
                idxs = list(carry[2])
                for u in range(UNROLL):
                    cu = col + u * _L
                    for r in range(rpw):
                        v = buf[r, pl.ds((i * UNROLL + u) * _L, _L)]
                        gt = v > vals[r]
                        vals[r] = jnp.where(gt, v, vals[r])
                        idxs[r] = jnp.where(gt, cu, idxs[r])
                return (col + UNROLL * _L, tuple(vals), tuple(idxs))

            col0 = lanes + col_base + c * cw
            _, mvt, mit = lax.fori_loop(
                0, cw // (_L * UNROLL), chunk_body,
                (col0, tuple(mv), tuple(mi)))
            mv = list(mvt)
            mi = list(mit)

        acc_i = zero_i
        acc_v = jnp.zeros((_L,), jnp.float32)
        for r in range(rpw):
            best_v = mv[r][0]
            best_i = mi[r][0]
            for lane in range(1, _L):
                v = mv[r][lane]
                i = mi[r][lane]
                take = (v > best_v) | ((v == best_v) & (i < best_i))
                best_v = jnp.where(take, v, best_v)
                best_i = jnp.where(take, i, best_i)
            acc_i = jnp.where(lanes == r, best_i, acc_i)
            acc_v = jnp.where(lanes == r, best_v, acc_v)

        idxv[...] = acc_i
        valv[...] = acc_v
        pltpu.sync_copy(idxv, idx_out.at[wid])
        pltpu.sync_copy(valv, val_out.at[wid])

    return sc_kernel(card_prob)


def _tc_pos_stage(choice_index, card_embed, pos_xt, pos_yt):
    B, D, P = pos_xt.shape
    V = card_embed.shape[0]
    bb = 16
    nbb = B // bb

    def body(idx_sref, embed_ref, x_ref, y_ref,
             card_ref, px_ref, pxp_ref, py_ref, pyp_ref):
        bi = pl.program_id(0)

        @pl.when(bi == 0)
        def _():
            def gather_one(b, _):
                i = idx_sref[b]
                card_ref[pl.ds(b, 1), :] = embed_ref[pl.ds(i, 1), :]
                return 0
            lax.fori_loop(0, B, gather_one, 0, unroll=8)

        card = card_ref[pl.ds(bi * bb, bb), :]

        def process(ref, out_i_ref, out_p_ref):
            blk = ref[...]
            lg = jnp.sum(blk * card[:, :, None], axis=1)
            bm = jnp.max(lg, axis=-1, keepdims=True)
            col = lax.broadcasted_iota(jnp.int32, (bb, P), 1)
            barg = jnp.min(jnp.where(lg == bm, col, 2**30),
                           axis=-1, keepdims=True)
            ssum = jnp.sum(jnp.exp(lg - bm), axis=-1, keepdims=True)
            out_i_ref[...] = barg
            out_p_ref[...] = 1.0 / ssum

        process(x_ref, px_ref, pxp_ref)
        process(y_ref, py_ref, pyp_ref)

    grid_spec = pltpu.PrefetchScalarGridSpec(
        num_scalar_prefetch=1,
        grid=(nbb,),
        in_specs=[
            pl.BlockSpec((V, D), lambda i, idx: (0, 0)),
            pl.BlockSpec((bb, D, P), lambda i, idx: (i, 0, 0)),
            pl.BlockSpec((bb, D, P), lambda i, idx: (i, 0, 0)),
        ],
        out_specs=[
            pl.BlockSpec((B, D), lambda i, idx: (0, 0)),
            pl.BlockSpec((bb, 1), lambda i, idx: (i, 0)),
            pl.BlockSpec((bb, 1), lambda i, idx: (i, 0)),
            pl.BlockSpec((bb, 1), lambda i, idx: (i, 0)),
            pl.BlockSpec((bb, 1), lambda i, idx: (i, 0)),
        ],
    )
    return pl.pallas_call(
        body,
        grid_spec=grid_spec,
        out_shape=[
            jax.ShapeDtypeStruct((B, D), jnp.float32),
            jax.ShapeDtypeStruct((B, 1), jnp.int32),
            jax.ShapeDtypeStruct((B, 1), jnp.float32),
            jax.ShapeDtypeStruct((B, 1), jnp.int32),
            jax.ShapeDtypeStruct((B, 1), jnp.float32),
        ],
    )(choice_index, card_embed, pos_xt, pos_yt)


def kernel(card_prob, pos_x_vector, pos_y_vector, card_embed, skip):
    B, V = card_prob.shape

    idx_nw, val_nw = _sc_card_argmax(card_prob, B, V)

    idx2 = idx_nw.reshape(_NW // 2, 2, _L)[:, :, :8]
    val2 = val_nw.reshape(_NW // 2, 2, _L)[:, :, :8]
    take1 = val2[:, 1] > val2[:, 0]
    raw_idx = jnp.where(take1, idx2[:, 1], idx2[:, 0]).reshape(B)
    raw_val = jnp.where(take1, val2[:, 1], val2[:, 0]).reshape(B)

    skip_flag = jnp.asarray(skip, jnp.int32) != 0
    choice_index = jnp.where(skip_flag, jnp.int32(0), raw_idx).astype(jnp.int32)
    card_prob_sel = jnp.where(skip_flag, card_prob[:, 0], raw_val)

    pos_xt = jnp.transpose(pos_x_vector, (0, 2, 1))
    pos_yt = jnp.transpose(pos_y_vector, (0, 2, 1))
    choice_card, px, pxp, py, pyp = _tc_pos_stage(
        choice_index, card_embed, pos_xt, pos_yt)

    return (
        choice_index,
        card_prob_sel,
        px[:, 0],
        pxp[:, 0],
        py[:, 0],
        pyp[:, 0],
        choice_card,
    )
